# baseline (speedup 1.0000x reference)
import functools

import jax
import jax.numpy as jnp
from jax import lax
from jax.experimental import pallas as pl
from jax.experimental.pallas import tpu as pltpu
from jax.experimental.pallas import tpu_sc as plsc

_D = 32
_B = 16384
_MARGIN = 1.0
_L = 16
_NW = 32
_BW = _B // _NW
_CH = 128
_NCH = _BW // _CH
_NVC = _CH // _L

_mesh = plsc.VectorSubcoreMesh(
    core_axis_name="c", subcore_axis_name="s", num_cores=2, num_subcores=16
)


@functools.partial(
    pl.kernel,
    out_type=jax.ShapeDtypeStruct((_NW, _L), jnp.float32),
    mesh=_mesh,
    scratch_types=(
        [pltpu.VMEM((_NCH, _CH), jnp.int32) for _ in range(6)]
        + [pltpu.VMEM((_CH, _CH), jnp.float32) for _ in range(6)]
        + [pltpu.VMEM((_L,), jnp.float32), pltpu.SemaphoreType.DMA]
    ),
    compiler_params=pltpu.CompilerParams(
        needs_layout_passes=False, use_tc_tiling_on_sc=True
    ),
)
def _transe_sc(ent_hbm, rel_hbm,
               p0, p1, p2, p3, p4, p5,
               out_hbm,
               ip0, ip1, ip2, ip3, ip4, ip5,
               b0, b1, b2, b3, b4, b5,
               loss_v, sem):
    wid = lax.axis_index("s") * 2 + lax.axis_index("c")

    p_hbms = (p0, p1, p2, p3, p4, p5)
    ip_refs = (ip0, ip1, ip2, ip3, ip4, ip5)
    bufs = (b0, b1, b2, b3, b4, b5)
    tables = (ent_hbm, rel_hbm, ent_hbm, ent_hbm, rel_hbm, ent_hbm)

    copies = [pltpu.async_copy(h.at[wid], r, sem)
              for h, r in zip(p_hbms, ip_refs)]
    for c in copies:
        c.wait()

    lane = lax.iota(jnp.int32, _L)
    zero = lax.broadcast(jnp.float32(0.0), (_L,))
    loss = zero

    for c in range(_NCH):
        gathers = [
            pltpu.async_copy(tab.at[iref.at[c]], bref, sem)
            for tab, iref, bref in zip(tables, ip_refs, bufs)
        ]
        for g in gathers:
            g.wait()

        def vreg_body(v, loss_sum):
            row = lane + v * _L
            acc_p = zero
            acc_n = zero
            for d in range(_D):
                col = lax.broadcast(jnp.int32(d), (_L,))
                hp = plsc.load_gather(b0, [row, col])
                rp = plsc.load_gather(b1, [row, col])
                tp = plsc.load_gather(b2, [row, col])
                acc_p = acc_p + jnp.abs(hp + rp - tp)
                hn = plsc.load_gather(b3, [row, col])
                rn = plsc.load_gather(b4, [row, col])
                tn = plsc.load_gather(b5, [row, col])
                acc_n = acc_n + jnp.abs(hn + rn - tn)
            hinge = jnp.maximum(acc_p - acc_n + jnp.float32(_MARGIN), zero)
            return loss_sum + hinge

        loss = lax.fori_loop(0, _NVC, vreg_body, loss)

    loss_v[...] = loss
    pltpu.sync_copy(loss_v, out_hbm.at[wid])


def kernel(positive_triplets, negative_triplets, entity_emb, relation_emb):
    cols = [
        arr[:, c].reshape(_NW, _NCH, _CH)
        for arr in (positive_triplets, negative_triplets)
        for c in range(3)
    ]
    ent_p = jnp.tile(entity_emb, (1, 128 // _D))
    rel_p = jnp.tile(relation_emb, (1, 128 // _D))
    partials = _transe_sc(ent_p, rel_p, *cols)
    return jnp.sum(partials) / jnp.float32(_B)


# speedup vs baseline: 1.4540x; 1.4540x over previous
import functools

import jax
import jax.numpy as jnp
from jax import lax
from jax.experimental import pallas as pl
from jax.experimental.pallas import tpu as pltpu
from jax.experimental.pallas import tpu_sc as plsc

_D = 32
_B = 16384
_MARGIN = 1.0
_L = 16
_NW = 32
_BW = _B // _NW
_CH = 128
_NCH = _BW // _CH
_NG = _BW // _L

_mesh = plsc.VectorSubcoreMesh(
    core_axis_name="c", subcore_axis_name="s", num_cores=2, num_subcores=16
)


@functools.partial(
    pl.kernel,
    out_type=jax.ShapeDtypeStruct((_NW, _L), jnp.float32),
    mesh=_mesh,
    scratch_types=(
        [pltpu.VMEM((_NCH, _CH), jnp.int32) for _ in range(6)]
        + [pltpu.VMEM((_BW, _D), jnp.float32) for _ in range(6)]
        + [pltpu.VMEM((_L,), jnp.float32), pltpu.SemaphoreType.DMA]
    ),
    compiler_params=pltpu.CompilerParams(
        needs_layout_passes=False, use_tc_tiling_on_sc=False
    ),
)
def _transe_sc(ent_hbm, rel_hbm,
               ph_hbm, pr_hbm, pt_hbm, nh_hbm, nr_hbm, nt_hbm,
               out_hbm,
               iph, ipr, ipt, inh, inr, int_,
               rph, rpr, rpt, rnh, rnr, rnt,
               loss_v, sem):
    wid = lax.axis_index("s") * 2 + lax.axis_index("c")

    idx_refs = (iph, ipr, ipt, inh, inr, int_)
    idx_hbms = (ph_hbm, pr_hbm, pt_hbm, nh_hbm, nr_hbm, nt_hbm)
    row_refs = (rph, rpr, rpt, rnh, rnr, rnt)
    tables = (ent_hbm, rel_hbm, ent_hbm, ent_hbm, rel_hbm, ent_hbm)

    copies = [
        pltpu.async_copy(h.at[wid], r, sem) for h, r in zip(idx_hbms, idx_refs)
    ]
    for c in copies:
        c.wait()

    gathers = []
    for tab, iref, rref in zip(tables, idx_refs, row_refs):
        for c in range(_NCH):
            gathers.append(
                pltpu.async_copy(
                    tab.at[iref.at[c]], rref.at[pl.ds(c * _CH, _CH)], sem
                )
            )
    for g in gathers:
        g.wait()

    lane = lax.iota(jnp.int32, _L)
    zero = lax.broadcast(jnp.float32(0.0), (_L,))

    def group_body(g, loss_sum):
        row_idx = lane + g * _L
        acc_p = zero
        acc_n = zero
        for d in range(_D):
            col = lax.broadcast(jnp.int32(d), (_L,))
            h_p = plsc.load_gather(rph, [row_idx, col])
            r_p = plsc.load_gather(rpr, [row_idx, col])
            t_p = plsc.load_gather(rpt, [row_idx, col])
            acc_p = acc_p + jnp.abs(h_p + r_p - t_p)
            h_n = plsc.load_gather(rnh, [row_idx, col])
            r_n = plsc.load_gather(rnr, [row_idx, col])
            t_n = plsc.load_gather(rnt, [row_idx, col])
            acc_n = acc_n + jnp.abs(h_n + r_n - t_n)
        hinge = jnp.maximum(acc_p - acc_n + jnp.float32(_MARGIN), zero)
        return loss_sum + hinge

    loss_sum = lax.fori_loop(0, _NG, group_body, zero)
    loss_v[...] = loss_sum
    pltpu.sync_copy(loss_v, out_hbm.at[wid])


def kernel(positive_triplets, negative_triplets, entity_emb, relation_emb):
    cols = [
        arr[:, c].reshape(_NW, _NCH, _CH)
        for arr in (positive_triplets, negative_triplets)
        for c in range(3)
    ]
    partials = _transe_sc(entity_emb, relation_emb, *cols)
    return jnp.sum(partials) / jnp.float32(_B)
